# baseline (speedup 1.0000x reference)
import jax
import jax.numpy as jnp
from jax import lax
from jax.experimental import pallas as pl
from jax.experimental.pallas import tpu as pltpu
from jax.experimental.pallas import tpu_sc as plsc

B, N, D_MODEL, H, K_NB = 2, 2048, 256, 8, 16
D_HEAD = D_MODEL // H
QT = 256
NT = (B * N) // QT
BIGF = 3.4e38
TW = 2 * D_MODEL + 128
TWW = 384

SC_NC, SC_NS = 2, 16
SC_NW = SC_NC * SC_NS
ROWS_PER_W = (N * K_NB) // SC_NW
CHUNK = 128
NCHUNK = ROWS_PER_W // CHUNK
NT_B = N // QT


def _projknn_body(pos_ref, posT_ref, feat_ref, wq_ref, wk_ref, wv_ref, wg_ref,
                  q_out, g_out, t_out, idx_out):
    b = pl.program_id(0)
    j = pl.program_id(1)

    f = feat_ref[0]
    q = jnp.dot(f, wq_ref[...], preferred_element_type=jnp.float32)
    k = jnp.dot(f, wk_ref[...], preferred_element_type=jnp.float32)
    v = jnp.dot(f, wv_ref[...], preferred_element_type=jnp.float32)
    g = jnp.dot(q, wg_ref[...], preferred_element_type=jnp.float32)
    q_out[0] = q
    g_out[0] = g

    pt = pos_ref[0]
    posrep = jnp.concatenate([pt] * H, axis=1)
    ka = jnp.concatenate([k, posrep, jnp.zeros((QT, 104), jnp.float32)],
                         axis=1)
    vb = jnp.concatenate([v, jnp.zeros((QT, 128), jnp.float32)], axis=1)
    ai = lax.bitcast_convert_type(ka, jnp.int32)
    bi = lax.bitcast_convert_type(vb, jnp.int32)
    hi = (ai + 0x8000) & jnp.int32(-65536)
    lo = lax.shift_right_logical(bi + 0x8000, 16) & jnp.int32(0xFFFF)
    t_out[0] = hi | lo

    pa = posT_ref[0]
    dx = pt[:, 0:1] - pa[0:1, :]
    dy = pt[:, 1:2] - pa[1:2, :]
    dz = pt[:, 2:3] - pa[2:3, :]
    dist = dx * dx + dy * dy + dz * dz

    colf = lax.broadcasted_iota(jnp.int32, (QT, N), 1).astype(jnp.float32)
    row_g = (j * QT
             + lax.broadcasted_iota(jnp.int32, (QT, 1), 0)).astype(jnp.float32)
    dist = jnp.where(colf == row_g, BIGF, dist)

    cols = []
    for _ in range(K_NB):
        m = jnp.min(dist, axis=1, keepdims=True)
        aminf = jnp.min(jnp.where(dist == m, colf, BIGF), axis=1,
                        keepdims=True)
        cols.append(aminf)
        dist = jnp.where(colf == aminf, BIGF, dist)
    idx = jnp.concatenate(cols, axis=1).astype(jnp.int32)
    idx_out[0] = idx.T


def _run_projknn(positions, positionsT, features, wq, wk, wv, wg):
    grid = (1, N // QT)
    spec_tile = lambda d: pl.BlockSpec((1, QT, d), lambda b, j: (b, j, 0))
    spec_full = lambda a, c: pl.BlockSpec((1, a, c), lambda b, j: (b, 0, 0))
    spec_w = pl.BlockSpec((D_MODEL, D_MODEL), lambda b, j: (0, 0))
    return pl.pallas_call(
        _projknn_body,
        grid=grid,
        in_specs=[
            spec_tile(3),
            spec_full(3, N),
            spec_tile(D_MODEL),
            spec_w, spec_w, spec_w,
            pl.BlockSpec((D_MODEL, 3 * H), lambda b, j: (0, 0)),
        ],
        out_specs=[
            spec_tile(D_MODEL),
            spec_tile(3 * H),
            spec_tile(TWW),
            pl.BlockSpec((1, K_NB, QT), lambda b, j: (j, 0, 0)),
        ],
        out_shape=[
            jax.ShapeDtypeStruct((1, N, D_MODEL), jnp.float32),
            jax.ShapeDtypeStruct((1, N, 3 * H), jnp.float32),
            jax.ShapeDtypeStruct((1, N, TWW), jnp.int32),
            jax.ShapeDtypeStruct((NT_B, K_NB, QT), jnp.int32),
        ],
    )(positions, positionsT, features, wq, wk, wv, wg)


def _sc_gather_body(ttab, idx_hbm, tout, idx_v, tbuf, gsem, wsem):
    wid = lax.axis_index("s") * SC_NC + lax.axis_index("c")
    base = wid * ROWS_PER_W
    pltpu.sync_copy(idx_hbm.at[pl.ds(base, ROWS_PER_W)], idx_v)
    isl0 = idx_v.at[pl.ds(0, CHUNK)]
    pltpu.async_copy(ttab.at[isl0], tbuf.at[0], gsem.at[0])
    for c in range(NCHUNK):
        r = c % 2
        if c + 1 < NCHUNK:
            rn = (c + 1) % 2
            if c + 1 >= 2:
                pltpu.make_async_copy(tbuf.at[rn], tout.at[pl.ds(0, CHUNK)],
                                      wsem.at[rn]).wait()
            isl = idx_v.at[pl.ds((c + 1) * CHUNK, CHUNK)]
            pltpu.async_copy(ttab.at[isl], tbuf.at[rn], gsem.at[rn])
        pltpu.make_async_copy(ttab.at[isl0], tbuf.at[r], gsem.at[r]).wait()
        o = base + c * CHUNK
        pltpu.async_copy(tbuf.at[r], tout.at[pl.ds(o, CHUNK)], wsem.at[r])
    for r in range(2):
        pltpu.make_async_copy(tbuf.at[r], tout.at[pl.ds(0, CHUNK)],
                              wsem.at[r]).wait()


def _run_sc_gather(ttab, idx_flat):
    R = N * K_NB
    mesh = plsc.VectorSubcoreMesh(core_axis_name="c", subcore_axis_name="s")
    fn = pl.kernel(
        _sc_gather_body,
        out_type=jax.ShapeDtypeStruct((R, TWW), jnp.int32),
        mesh=mesh,
        scratch_types=[
            pltpu.VMEM((ROWS_PER_W,), jnp.int32),
            pltpu.VMEM((2, CHUNK, TWW), jnp.int32),
            pltpu.SemaphoreType.DMA((2,)),
            pltpu.SemaphoreType.DMA((2,)),
        ],
    )
    return fn(ttab, idx_flat)


def _attend_body(q_ref, g_ref, gt_ref, feat_ref, wout_ref,
                 bout_ref, gam_ref, bet_ref, seg_ref, sum_ref, exp_ref,
                 out_ref):
    q = q_ref[...]
    g = g_ref[...]
    qg = jnp.concatenate([q, -g], axis=1)
    seg = seg_ref[...]
    summat = sum_ref[...]
    expm = exp_ref[...]

    scols = []
    for kk in range(K_NB):
        w = gt_ref[pl.ds(kk * QT, QT), 0:D_MODEL + 3 * H]
        comb = lax.bitcast_convert_type(w & jnp.int32(-65536), jnp.float32)
        s = jnp.dot(qg * comb, seg, preferred_element_type=jnp.float32)
        scols.append(s)
    s2 = jnp.concatenate(scols, axis=1)
    s2 = s2 - jnp.max(s2)
    e2 = jnp.exp(s2)
    dent = jnp.dot(e2, summat, preferred_element_type=jnp.float32)
    w2 = e2 * (1.0 / dent)
    acc = jnp.zeros((QT, D_MODEL), jnp.float32)
    for kk in range(K_NB):
        wexp = jnp.dot(w2[:, kk * H:(kk + 1) * H], expm,
                       preferred_element_type=jnp.float32)
        wv = gt_ref[pl.ds(kk * QT, QT), 0:D_MODEL]
        vsl = lax.bitcast_convert_type(lax.shift_left(wv, 16), jnp.float32)
        acc = acc + wexp * vsl
    out = jnp.dot(acc, wout_ref[...], preferred_element_type=jnp.float32)
    x = out + bout_ref[...] + feat_ref[...]
    mu = jnp.mean(x, axis=1, keepdims=True)
    xc = x - mu
    var = jnp.mean(xc * xc, axis=1, keepdims=True)
    out_ref[...] = xc * jax.lax.rsqrt(var + 1e-5) * gam_ref[...] + bet_ref[...]


def _run_attend(q, g, gt, feats, wout, bout, gam, bet, seg, summat, expm):
    t = lambda d: pl.BlockSpec((QT, d), lambda i: (i, 0))
    big = lambda d: pl.BlockSpec((QT * K_NB, d), lambda i: (i, 0))
    full = lambda a, c: pl.BlockSpec((a, c), lambda i: (0, 0))
    return pl.pallas_call(
        _attend_body,
        grid=(NT_B,),
        in_specs=[
            t(D_MODEL), t(3 * H), big(TWW),
            t(D_MODEL), full(D_MODEL, D_MODEL), full(1, D_MODEL),
            full(1, D_MODEL), full(1, D_MODEL), full(D_MODEL + 3 * H, H),
            full(K_NB * H, K_NB * H), full(H, D_MODEL),
        ],
        out_specs=t(D_MODEL),
        out_shape=jax.ShapeDtypeStruct((N, D_MODEL), jnp.float32),
    )(q, g, gt, feats, wout, bout, gam, bet, seg, summat, expm)


def kernel(positions, features, Wq, Wk, Wv, Wout, b_out, Wpos, b_pos,
           temperature, ln_gamma, ln_beta):
    wg = jnp.kron(jnp.eye(H, dtype=jnp.float32), Wpos[:, :D_HEAD].T)
    positionsT = positions.transpose(0, 2, 1)

    eyeh = jnp.eye(H, dtype=jnp.float32)
    seg = jnp.concatenate(
        [jnp.kron(eyeh, jnp.ones((D_HEAD, 1), jnp.float32)),
         jnp.kron(eyeh, jnp.ones((3, 1), jnp.float32))],
        axis=0) / temperature[0]
    summat = jnp.kron(jnp.ones((K_NB, K_NB), jnp.float32), eyeh)
    expm = jnp.kron(eyeh, jnp.ones((1, D_HEAD), jnp.float32))

    outs = []
    for b in range(B):
        q, g, tab, nbr = _run_projknn(positions[b:b + 1],
                                      positionsT[b:b + 1],
                                      features[b:b + 1], Wq, Wk, Wv, wg)
        idx_flat = nbr.reshape(N * K_NB)
        gt = _run_sc_gather(tab.reshape(N, TWW), idx_flat)
        outs.append(_run_attend(q.reshape(N, D_MODEL), g.reshape(N, 3 * H),
                                gt, features[b], Wout,
                                b_out.reshape(1, D_MODEL),
                                ln_gamma.reshape(1, D_MODEL),
                                ln_beta.reshape(1, D_MODEL),
                                seg, summat, expm))
    return jnp.stack(outs)


# speedup vs baseline: 1.0019x; 1.0019x over previous
import jax
import jax.numpy as jnp
from jax import lax
from jax.experimental import pallas as pl
from jax.experimental.pallas import tpu as pltpu
from jax.experimental.pallas import tpu_sc as plsc

B, N, D_MODEL, H, K_NB = 2, 2048, 256, 8, 16
D_HEAD = D_MODEL // H
QT = 256
BIGF = 3.4e38
TW = 2 * D_MODEL + 128
TWW = 384

SC_NC, SC_NS = 2, 16
SC_NW = SC_NC * SC_NS
ROWS_PER_W = (N * K_NB) // SC_NW
CHUNK = 64
NCHUNK = ROWS_PER_W // CHUNK
NT_B = N // QT


def _projknn_body(pos_ref, posT_ref, feat_ref, wq_ref, wk_ref, wv_ref, wg_ref,
                  q_out, g_out, t_out, idx_out):
    j = pl.program_id(1)

    f = feat_ref[0]
    q = jnp.dot(f, wq_ref[...], preferred_element_type=jnp.float32)
    k = jnp.dot(f, wk_ref[...], preferred_element_type=jnp.float32)
    v = jnp.dot(f, wv_ref[...], preferred_element_type=jnp.float32)
    g = jnp.dot(q, wg_ref[...], preferred_element_type=jnp.float32)
    q_out[0] = q
    g_out[0] = g

    pt = pos_ref[0]
    posrep = jnp.concatenate([pt] * H, axis=1)
    ka = jnp.concatenate([k, posrep, jnp.zeros((QT, 104), jnp.float32)],
                         axis=1)
    vb = jnp.concatenate([v, jnp.zeros((QT, 128), jnp.float32)], axis=1)
    ai = lax.bitcast_convert_type(ka, jnp.int32)
    bi = lax.bitcast_convert_type(vb, jnp.int32)
    hi = (ai + 0x8000) & jnp.int32(-65536)
    lo = lax.shift_right_logical(bi + 0x8000, 16) & jnp.int32(0xFFFF)
    t_out[0] = hi | lo

    pa = posT_ref[0]
    dx = pt[:, 0:1] - pa[0:1, :]
    dy = pt[:, 1:2] - pa[1:2, :]
    dz = pt[:, 2:3] - pa[2:3, :]
    dist = dx * dx + dy * dy + dz * dz

    colf = lax.broadcasted_iota(jnp.int32, (QT, N), 1).astype(jnp.float32)
    row_g = (j * QT
             + lax.broadcasted_iota(jnp.int32, (QT, 1), 0)).astype(jnp.float32)
    dist = jnp.where(colf == row_g, BIGF, dist)

    cols = []
    for _ in range(K_NB):
        m = jnp.min(dist, axis=1, keepdims=True)
        aminf = jnp.min(jnp.where(dist == m, colf, BIGF), axis=1,
                        keepdims=True)
        cols.append(aminf)
        dist = jnp.where(colf == aminf, BIGF, dist)
    idx = jnp.concatenate(cols, axis=1).astype(jnp.int32)
    idx_out[0] = idx.T


def _run_projknn(positions, positionsT, features, wq, wk, wv, wg):
    grid = (1, N // QT)
    spec_tile = lambda d: pl.BlockSpec((1, QT, d), lambda b, j: (b, j, 0))
    spec_full = lambda a, c: pl.BlockSpec((1, a, c), lambda b, j: (b, 0, 0))
    spec_w = pl.BlockSpec((D_MODEL, D_MODEL), lambda b, j: (0, 0))
    return pl.pallas_call(
        _projknn_body,
        grid=grid,
        in_specs=[
            spec_tile(3),
            spec_full(3, N),
            spec_tile(D_MODEL),
            spec_w, spec_w, spec_w,
            pl.BlockSpec((D_MODEL, 3 * H), lambda b, j: (0, 0)),
        ],
        out_specs=[
            spec_tile(D_MODEL),
            spec_tile(3 * H),
            spec_tile(TWW),
            pl.BlockSpec((1, K_NB, QT), lambda b, j: (j, 0, 0)),
        ],
        out_shape=[
            jax.ShapeDtypeStruct((1, N, D_MODEL), jnp.float32),
            jax.ShapeDtypeStruct((1, N, 3 * H), jnp.float32),
            jax.ShapeDtypeStruct((1, N, TWW), jnp.int32),
            jax.ShapeDtypeStruct((NT_B, K_NB, QT), jnp.int32),
        ],
    )(positions, positionsT, features, wq, wk, wv, wg)


def _sc_gather_body(ttab, idx_hbm, tout, idx_v, tbuf, gsem, wsem):
    wid = lax.axis_index("s") * SC_NC + lax.axis_index("c")
    base = wid * ROWS_PER_W
    pltpu.sync_copy(idx_hbm.at[pl.ds(base, ROWS_PER_W)], idx_v)
    isl0 = idx_v.at[pl.ds(0, CHUNK)]
    pltpu.async_copy(ttab.at[isl0], tbuf.at[0], gsem.at[0])
    for c in range(NCHUNK):
        r = c % 2
        if c + 1 < NCHUNK:
            rn = (c + 1) % 2
            if c + 1 >= 2:
                pltpu.make_async_copy(tbuf.at[rn], tout.at[pl.ds(0, CHUNK)],
                                      wsem.at[rn]).wait()
            isl = idx_v.at[pl.ds((c + 1) * CHUNK, CHUNK)]
            pltpu.async_copy(ttab.at[isl], tbuf.at[rn], gsem.at[rn])
        pltpu.make_async_copy(ttab.at[isl0], tbuf.at[r], gsem.at[r]).wait()
        o = base + c * CHUNK
        pltpu.async_copy(tbuf.at[r], tout.at[pl.ds(o, CHUNK)], wsem.at[r])
    for r in range(2):
        pltpu.make_async_copy(tbuf.at[r], tout.at[pl.ds(0, CHUNK)],
                              wsem.at[r]).wait()


def _run_sc_gather(ttab, idx_flat):
    R = N * K_NB
    mesh = plsc.VectorSubcoreMesh(core_axis_name="c", subcore_axis_name="s")
    fn = pl.kernel(
        _sc_gather_body,
        out_type=jax.ShapeDtypeStruct((R, TWW), jnp.int32),
        mesh=mesh,
        scratch_types=[
            pltpu.VMEM((ROWS_PER_W,), jnp.int32),
            pltpu.VMEM((2, CHUNK, TWW), jnp.int32),
            pltpu.SemaphoreType.DMA((2,)),
            pltpu.SemaphoreType.DMA((2,)),
        ],
    )
    return fn(ttab, idx_flat)


def _attend_body(q_ref, g_ref, gt_ref, feat_ref, wout_ref,
                 bout_ref, gam_ref, bet_ref, seg_ref, sum_ref, exp_ref,
                 out_ref):
    q = q_ref[...]
    g = g_ref[...]
    qg = jnp.concatenate([q, -g], axis=1)
    seg = seg_ref[...]
    summat = sum_ref[...]
    expm = exp_ref[...]

    scols = []
    for kk in range(K_NB):
        w = gt_ref[pl.ds(kk * QT, QT), 0:D_MODEL + 3 * H]
        comb = lax.bitcast_convert_type(w & jnp.int32(-65536), jnp.float32)
        s = jnp.dot(qg * comb, seg, preferred_element_type=jnp.float32)
        scols.append(s)
    s2 = jnp.concatenate(scols, axis=1)
    s2 = s2 - jnp.max(s2)
    e2 = jnp.exp(s2)
    dent = jnp.dot(e2, summat, preferred_element_type=jnp.float32)
    w2 = e2 * (1.0 / dent)
    acc = jnp.zeros((QT, D_MODEL), jnp.float32)
    for kk in range(K_NB):
        wexp = jnp.dot(w2[:, kk * H:(kk + 1) * H], expm,
                       preferred_element_type=jnp.float32)
        wv = gt_ref[pl.ds(kk * QT, QT), 0:D_MODEL]
        vsl = lax.bitcast_convert_type(lax.shift_left(wv, 16), jnp.float32)
        acc = acc + wexp * vsl
    out = jnp.dot(acc, wout_ref[...], preferred_element_type=jnp.float32)
    x = out + bout_ref[...] + feat_ref[...]
    mu = jnp.mean(x, axis=1, keepdims=True)
    xc = x - mu
    var = jnp.mean(xc * xc, axis=1, keepdims=True)
    out_ref[...] = xc * jax.lax.rsqrt(var + 1e-5) * gam_ref[...] + bet_ref[...]


def _run_attend(q, g, gt, feats, wout, bout, gam, bet, seg, summat, expm):
    t = lambda d: pl.BlockSpec((QT, d), lambda i: (i, 0))
    big = lambda d: pl.BlockSpec((QT * K_NB, d), lambda i: (i, 0))
    full = lambda a, c: pl.BlockSpec((a, c), lambda i: (0, 0))
    return pl.pallas_call(
        _attend_body,
        grid=(NT_B,),
        in_specs=[
            t(D_MODEL), t(3 * H), big(TWW),
            t(D_MODEL), full(D_MODEL, D_MODEL), full(1, D_MODEL),
            full(1, D_MODEL), full(1, D_MODEL), full(D_MODEL + 3 * H, H),
            full(K_NB * H, K_NB * H), full(H, D_MODEL),
        ],
        out_specs=t(D_MODEL),
        out_shape=jax.ShapeDtypeStruct((N, D_MODEL), jnp.float32),
    )(q, g, gt, feats, wout, bout, gam, bet, seg, summat, expm)


def kernel(positions, features, Wq, Wk, Wv, Wout, b_out, Wpos, b_pos,
           temperature, ln_gamma, ln_beta):
    wg = jnp.kron(jnp.eye(H, dtype=jnp.float32), Wpos[:, :D_HEAD].T)
    positionsT = positions.transpose(0, 2, 1)

    eyeh = jnp.eye(H, dtype=jnp.float32)
    seg = jnp.concatenate(
        [jnp.kron(eyeh, jnp.ones((D_HEAD, 1), jnp.float32)),
         jnp.kron(eyeh, jnp.ones((3, 1), jnp.float32))],
        axis=0) / temperature[0]
    summat = jnp.kron(jnp.ones((K_NB, K_NB), jnp.float32), eyeh)
    expm = jnp.kron(eyeh, jnp.ones((1, D_HEAD), jnp.float32))

    outs = []
    for b in range(B):
        q, g, tab, nbr = _run_projknn(positions[b:b + 1],
                                      positionsT[b:b + 1],
                                      features[b:b + 1], Wq, Wk, Wv, wg)
        idx_flat = nbr.reshape(N * K_NB)
        gt = _run_sc_gather(tab.reshape(N, TWW), idx_flat)
        outs.append(_run_attend(q.reshape(N, D_MODEL), g.reshape(N, 3 * H),
                                gt, features[b], Wout,
                                b_out.reshape(1, D_MODEL),
                                ln_gamma.reshape(1, D_MODEL),
                                ln_beta.reshape(1, D_MODEL),
                                seg, summat, expm))
    return jnp.stack(outs)


# speedup vs baseline: 1.0246x; 1.0226x over previous
import jax
import jax.numpy as jnp
from jax import lax
from jax.experimental import pallas as pl
from jax.experimental.pallas import tpu as pltpu
from jax.experimental.pallas import tpu_sc as plsc

B, N, D_MODEL, H, K_NB = 2, 2048, 256, 8, 16
D_HEAD = D_MODEL // H
QT = 256
BIGF = 3.4e38
TW = 2 * D_MODEL + 128
TWW = 384

SC_NC, SC_NS = 2, 16
SC_NW = SC_NC * SC_NS
ROWS_PER_W = (N * K_NB) // SC_NW
CHUNK = 64
NCHUNK = ROWS_PER_W // CHUNK
NT_B = N // QT


def _projknn_body(pos_ref, posT_ref, feat_ref, wq_ref, wk_ref, wv_ref, wg_ref,
                  q_out, g_out, t_out, idx_out):
    j = pl.program_id(0)

    f = feat_ref[0]
    q = jnp.dot(f, wq_ref[...], preferred_element_type=jnp.float32)
    k = jnp.dot(f, wk_ref[...], preferred_element_type=jnp.float32)
    v = jnp.dot(f, wv_ref[...], preferred_element_type=jnp.float32)
    g = jnp.dot(q, wg_ref[...], preferred_element_type=jnp.float32)
    q_out[0] = q
    g_out[0] = g

    pt = pos_ref[0]
    posrep = jnp.concatenate([pt] * H, axis=1)
    ka = jnp.concatenate([k, posrep, jnp.zeros((QT, 104), jnp.float32)],
                         axis=1)
    vb = jnp.concatenate([v, jnp.zeros((QT, 128), jnp.float32)], axis=1)
    ai = lax.bitcast_convert_type(ka, jnp.int32)
    bi = lax.bitcast_convert_type(vb, jnp.int32)
    hi = (ai + 0x8000) & jnp.int32(-65536)
    lo = lax.shift_right_logical(bi + 0x8000, 16) & jnp.int32(0xFFFF)
    t_out[0] = hi | lo

    pa = posT_ref[0]
    dx = pt[:, 0:1] - pa[0:1, :]
    dy = pt[:, 1:2] - pa[1:2, :]
    dz = pt[:, 2:3] - pa[2:3, :]
    dist = dx * dx + dy * dy + dz * dz

    colf = lax.broadcasted_iota(jnp.int32, (QT, N), 1).astype(jnp.float32)
    row_g = (j * QT
             + lax.broadcasted_iota(jnp.int32, (QT, 1), 0)).astype(jnp.float32)
    dist = jnp.where(colf == row_g, BIGF, dist)

    cols = []
    for _ in range(K_NB):
        m = jnp.min(dist, axis=1, keepdims=True)
        aminf = jnp.min(jnp.where(dist == m, colf, BIGF), axis=1,
                        keepdims=True)
        cols.append(aminf)
        dist = jnp.where(colf == aminf, BIGF, dist)
    idx = jnp.concatenate(cols, axis=1).astype(jnp.int32)
    idx_out[0] = idx.T


def _run_projknn(positions, positionsT, features, wq, wk, wv, wg, b):
    grid = (N // QT,)
    spec_tile = lambda d: pl.BlockSpec((1, QT, d), lambda j: (b, j, 0))
    spec_out = lambda d: pl.BlockSpec((1, QT, d), lambda j: (0, j, 0))
    spec_full = lambda a, c: pl.BlockSpec((1, a, c), lambda j: (b, 0, 0))
    spec_w = pl.BlockSpec((D_MODEL, D_MODEL), lambda j: (0, 0))
    return pl.pallas_call(
        _projknn_body,
        grid=grid,
        in_specs=[
            spec_tile(3),
            spec_full(3, N),
            spec_tile(D_MODEL),
            spec_w, spec_w, spec_w,
            pl.BlockSpec((D_MODEL, 3 * H), lambda j: (0, 0)),
        ],
        out_specs=[
            spec_out(D_MODEL),
            spec_out(3 * H),
            spec_out(TWW),
            pl.BlockSpec((1, K_NB, QT), lambda j: (j, 0, 0)),
        ],
        out_shape=[
            jax.ShapeDtypeStruct((1, N, D_MODEL), jnp.float32),
            jax.ShapeDtypeStruct((1, N, 3 * H), jnp.float32),
            jax.ShapeDtypeStruct((1, N, TWW), jnp.int32),
            jax.ShapeDtypeStruct((NT_B, K_NB, QT), jnp.int32),
        ],
    )(positions, positionsT, features, wq, wk, wv, wg)


def _sc_gather_body(ttab, idx_hbm, tout, idx_v, tbuf, gsem, wsem):
    wid = lax.axis_index("s") * SC_NC + lax.axis_index("c")
    base = wid * ROWS_PER_W
    pltpu.sync_copy(idx_hbm.at[pl.ds(base, ROWS_PER_W)], idx_v)
    isl0 = idx_v.at[pl.ds(0, CHUNK)]
    pltpu.async_copy(ttab.at[isl0], tbuf.at[0], gsem.at[0])
    for c in range(NCHUNK):
        r = c % 2
        if c + 1 < NCHUNK:
            rn = (c + 1) % 2
            if c + 1 >= 2:
                pltpu.make_async_copy(tbuf.at[rn], tout.at[pl.ds(0, CHUNK)],
                                      wsem.at[rn]).wait()
            isl = idx_v.at[pl.ds((c + 1) * CHUNK, CHUNK)]
            pltpu.async_copy(ttab.at[isl], tbuf.at[rn], gsem.at[rn])
        pltpu.make_async_copy(ttab.at[isl0], tbuf.at[r], gsem.at[r]).wait()
        o = base + c * CHUNK
        pltpu.async_copy(tbuf.at[r], tout.at[pl.ds(o, CHUNK)], wsem.at[r])
    for r in range(2):
        pltpu.make_async_copy(tbuf.at[r], tout.at[pl.ds(0, CHUNK)],
                              wsem.at[r]).wait()


def _run_sc_gather(ttab, idx_flat):
    R = N * K_NB
    mesh = plsc.VectorSubcoreMesh(core_axis_name="c", subcore_axis_name="s")
    fn = pl.kernel(
        _sc_gather_body,
        out_type=jax.ShapeDtypeStruct((R, TWW), jnp.int32),
        mesh=mesh,
        scratch_types=[
            pltpu.VMEM((ROWS_PER_W,), jnp.int32),
            pltpu.VMEM((2, CHUNK, TWW), jnp.int32),
            pltpu.SemaphoreType.DMA((2,)),
            pltpu.SemaphoreType.DMA((2,)),
        ],
    )
    return fn(ttab, idx_flat)


def _attend_body(q_ref, g_ref, gt_ref, feat_ref, wout_ref,
                 bout_ref, gam_ref, bet_ref, seg_ref, sum_ref, exp_ref,
                 out_ref):
    q = q_ref[...]
    g = g_ref[...]
    qg = jnp.concatenate([q, -g], axis=1)
    seg = seg_ref[...]
    summat = sum_ref[...]
    expm = exp_ref[...]

    scols = []
    for kk in range(K_NB):
        w = gt_ref[pl.ds(kk * QT, QT), 0:D_MODEL + 3 * H]
        comb = lax.bitcast_convert_type(w & jnp.int32(-65536), jnp.float32)
        s = jnp.dot(qg * comb, seg, preferred_element_type=jnp.float32)
        scols.append(s)
    s2 = jnp.concatenate(scols, axis=1)
    s2 = s2 - jnp.max(s2)
    e2 = jnp.exp(s2)
    dent = jnp.dot(e2, summat, preferred_element_type=jnp.float32)
    w2 = e2 * (1.0 / dent)
    acc = jnp.zeros((QT, D_MODEL), jnp.float32)
    for kk in range(K_NB):
        wexp = jnp.dot(w2[:, kk * H:(kk + 1) * H], expm,
                       preferred_element_type=jnp.float32)
        wv = gt_ref[pl.ds(kk * QT, QT), 0:D_MODEL]
        vsl = lax.bitcast_convert_type(lax.shift_left(wv, 16), jnp.float32)
        acc = acc + wexp * vsl
    out = jnp.dot(acc, wout_ref[...], preferred_element_type=jnp.float32)
    x = out + bout_ref[...] + feat_ref[0]
    mu = jnp.mean(x, axis=1, keepdims=True)
    xc = x - mu
    var = jnp.mean(xc * xc, axis=1, keepdims=True)
    out_ref[...] = xc * jax.lax.rsqrt(var + 1e-5) * gam_ref[...] + bet_ref[...]


def _run_attend(q, g, gt, feats, wout, bout, gam, bet, seg, summat, expm, b):
    t = lambda d: pl.BlockSpec((QT, d), lambda i: (i, 0))
    big = lambda d: pl.BlockSpec((QT * K_NB, d), lambda i: (i, 0))
    full = lambda a, c: pl.BlockSpec((a, c), lambda i: (0, 0))
    fspec = pl.BlockSpec((1, QT, D_MODEL), lambda i: (b, i, 0))
    return pl.pallas_call(
        _attend_body,
        grid=(NT_B,),
        in_specs=[
            t(D_MODEL), t(3 * H), big(TWW),
            fspec, full(D_MODEL, D_MODEL), full(1, D_MODEL),
            full(1, D_MODEL), full(1, D_MODEL), full(D_MODEL + 3 * H, H),
            full(K_NB * H, K_NB * H), full(H, D_MODEL),
        ],
        out_specs=t(D_MODEL),
        out_shape=jax.ShapeDtypeStruct((N, D_MODEL), jnp.float32),
    )(q, g, gt, feats, wout, bout, gam, bet, seg, summat, expm)


def kernel(positions, features, Wq, Wk, Wv, Wout, b_out, Wpos, b_pos,
           temperature, ln_gamma, ln_beta):
    wg = jnp.kron(jnp.eye(H, dtype=jnp.float32), Wpos[:, :D_HEAD].T)
    positionsT = positions.transpose(0, 2, 1)

    eyeh = jnp.eye(H, dtype=jnp.float32)
    seg = jnp.concatenate(
        [jnp.kron(eyeh, jnp.ones((D_HEAD, 1), jnp.float32)),
         jnp.kron(eyeh, jnp.ones((3, 1), jnp.float32))],
        axis=0) / temperature[0]
    summat = jnp.kron(jnp.ones((K_NB, K_NB), jnp.float32), eyeh)
    expm = jnp.kron(eyeh, jnp.ones((1, D_HEAD), jnp.float32))

    outs = []
    for b in range(B):
        q, g, tab, nbr = _run_projknn(positions, positionsT, features,
                                      Wq, Wk, Wv, wg, b)
        idx_flat = nbr.reshape(N * K_NB)
        gt = _run_sc_gather(tab.reshape(N, TWW), idx_flat)
        outs.append(_run_attend(q.reshape(N, D_MODEL), g.reshape(N, 3 * H),
                                gt, features, Wout,
                                b_out.reshape(1, D_MODEL),
                                ln_gamma.reshape(1, D_MODEL),
                                ln_beta.reshape(1, D_MODEL),
                                seg, summat, expm, b))
    return jnp.stack(outs)
